# baseline (speedup 1.0000x reference)
import functools

import jax
import jax.numpy as jnp
from jax import lax
from jax.experimental import pallas as pl
from jax.experimental.pallas import tpu as pltpu
from jax.experimental.pallas import tpu_sc as plsc

N = 10000
E = 320000
D = 128
THRESH = 0.1
NC = 2
NS = 16
NW = NC * NS
EPW = E // NW
CH = 80
NCH = EPW // CH
NPAD = 10240
ZB = NPAD // NS

_mesh = plsc.VectorSubcoreMesh(core_axis_name="c", subcore_axis_name="s")


def _zero_vec(ref, nwords):
    def body(i, _):
        ref[pl.ds(i * 16, 16)] = jnp.zeros((16,), jnp.float32)
        return 0
    lax.fori_loop(0, nwords // 16, body, 0)


@functools.partial(
    pl.kernel,
    out_type=(
        jax.ShapeDtypeStruct((NW, NCH, CH), jnp.float32),
        jax.ShapeDtypeStruct((NC, NPAD), jnp.float32),
    ),
    mesh=_mesh,
    compiler_params=pltpu.CompilerParams(needs_layout_passes=False),
    scratch_types=(
        pltpu.VMEM((NCH, CH), jnp.int32),
        pltpu.VMEM((NCH, CH), jnp.int32),
        pltpu.VMEM((2, CH), jnp.float32),
        pltpu.VMEM((2, CH, D), jnp.float32),
        pltpu.VMEM((2, CH, D), jnp.float32),
        pltpu.VMEM((16, 17), jnp.float32),
        pltpu.VMEM((ZB,), jnp.float32),
        pltpu.VMEM_SHARED((NPAD,), jnp.float32),
        pltpu.SemaphoreType.DMA,
        pltpu.SemaphoreType.DMA,
        pltpu.SemaphoreType.DMA,
        pltpu.SemaphoreType.DMA,
        pltpu.SemaphoreType.DMA,
        pltpu.SemaphoreType.DMA,
        pltpu.SemaphoreType.DMA,
        pltpu.SemaphoreType.DMA,
    ),
)
def _sc_attention(xn, row3, col3, att_out, rs_out, rixs, cixs, attv,
                  arows, brows, tbuf, zbuf, rssh, sa0, sb0, sa1, sb1, so0, so1,
                  ss0, ss1):
    c = lax.axis_index("c")
    s = lax.axis_index("s")
    w = s * NC + c

    _zero_vec(zbuf, ZB)
    pltpu.sync_copy(zbuf, rssh.at[pl.ds(s * ZB, ZB)])
    plsc.subcore_barrier()

    pltpu.sync_copy(row3.at[w], rixs)
    pltpu.sync_copy(col3.at[w], cixs)

    lanes = lax.iota(jnp.int32, 16)
    sems = ((sa0, sb0, so0, ss0), (sa1, sb1, so1, ss1))

    def issue(g, b):
        pltpu.async_copy(xn.at[rixs.at[g]], arows.at[b], sems[b][0])
        pltpu.async_copy(xn.at[cixs.at[g]], brows.at[b], sems[b][1])

    issue(0, 0)

    def do_chunk(g, b):
        ar = arows.at[b]
        br = brows.at[b]
        ab = attv.at[b]
        pltpu.make_async_copy(xn.at[rixs.at[g]], ar, sems[b][0]).wait()
        pltpu.make_async_copy(xn.at[cixs.at[g]], br, sems[b][1]).wait()

        def grp16(i, _):
            for l in range(16):
                e = i * 16 + l
                acc = ar[e, pl.ds(0, 16)] * br[e, pl.ds(0, 16)]
                for j in range(1, 8):
                    acc = acc + ar[e, pl.ds(16 * j, 16)] * br[e, pl.ds(16 * j, 16)]
                tbuf[l, pl.ds(0, 16)] = acc
            cols = [plsc.load_gather(tbuf, [lanes, jnp.full((16,), j, jnp.int32)])
                    for j in range(16)]
            while len(cols) > 1:
                cols = [cols[k] + cols[k + 1] for k in range(0, len(cols), 2)]
            v = cols[0]
            ab[pl.ds(i * 16, 16)] = jnp.where(v < THRESH, 0.0, v)
            return 0
        lax.fori_loop(0, CH // 16, grp16, 0)

        pltpu.async_copy(ab, att_out.at[w, g], sems[b][2])
        pltpu.async_copy(ab, rssh.at[rixs.at[g]], sems[b][3], add=True)

    def body(g, _):
        for par in range(2):
            @pl.when(lax.rem(g, 2) == par)
            def _():
                @pl.when(g + 1 < NCH)
                def _():
                    issue(g + 1, 1 - par)
                @pl.when(g >= 2)
                def _():
                    pltpu.make_async_copy(
                        attv.at[par], att_out.at[w, g], sems[par][2]).wait()
                    pltpu.make_async_copy(
                        attv.at[par], rssh.at[rixs.at[g]], sems[par][3]).wait()
                do_chunk(g, par)
        return 0
    lax.fori_loop(0, NCH, body, 0)
    for par in range(2):
        pltpu.make_async_copy(attv.at[par], att_out.at[w, 0],
                              sems[par][2]).wait()
        pltpu.make_async_copy(attv.at[par], rssh.at[rixs.at[0]],
                              sems[par][3]).wait()

    plsc.subcore_barrier()
    pltpu.sync_copy(rssh.at[pl.ds(s * ZB, ZB)], rs_out.at[c, pl.ds(s * ZB, ZB)])


@functools.partial(
    pl.kernel,
    out_type=jax.ShapeDtypeStruct((NC, NPAD), jnp.float32),
    mesh=_mesh,
    compiler_params=pltpu.CompilerParams(needs_layout_passes=False),
    scratch_types=(
        pltpu.VMEM((NCH, CH), jnp.int32),
        pltpu.VMEM((NCH, CH), jnp.int32),
        pltpu.VMEM((NCH, CH), jnp.float32),
        pltpu.VMEM((2, CH), jnp.float32),
        pltpu.VMEM((NPAD,), jnp.float32),
        pltpu.VMEM((NPAD,), jnp.float32),
        pltpu.VMEM((ZB,), jnp.float32),
        pltpu.VMEM_SHARED((NPAD,), jnp.float32),
        pltpu.SemaphoreType.DMA,
        pltpu.SemaphoreType.DMA,
    ),
)
def _sc_degree(att3, row3, col3, rsparts, deg_out, rixs, cixs, atts, uv,
               irtab, rstmp, zbuf, degsh, su0, su1):
    c = lax.axis_index("c")
    s = lax.axis_index("s")
    w = s * NC + c

    pltpu.sync_copy(rsparts.at[0], irtab)
    pltpu.sync_copy(rsparts.at[1], rstmp)

    def mkir(i, _):
        sl = pl.ds(i * 16, 16)
        rs = irtab[sl] + rstmp[sl]
        irtab[sl] = 1.0 / jnp.maximum(rs, 1e-12)
        return 0
    lax.fori_loop(0, NPAD // 16, mkir, 0)

    _zero_vec(zbuf, ZB)
    pltpu.sync_copy(zbuf, degsh.at[pl.ds(s * ZB, ZB)])
    pltpu.sync_copy(row3.at[w], rixs)
    pltpu.sync_copy(col3.at[w], cixs)
    pltpu.sync_copy(att3.at[w], atts)
    plsc.subcore_barrier()

    sems = (su0, su1)

    def chunk(g, _):
        for par in range(2):
            @pl.when(lax.rem(g, 2) == par)
            def _():
                ub = uv.at[par]
                @pl.when(g >= 2)
                def _():
                    pltpu.make_async_copy(
                        ub, degsh.at[cixs.at[g]], sems[par]).wait()

                def grp(i, _):
                    sl = pl.ds(i * 16, 16)
                    r16 = rixs[g, sl]
                    ir = plsc.load_gather(irtab, [r16])
                    ub[sl] = atts[g, sl] * ir
                    return 0
                lax.fori_loop(0, CH // 16, grp, 0)

                pltpu.async_copy(ub, degsh.at[cixs.at[g]], sems[par], add=True)
        return 0
    lax.fori_loop(0, NCH, chunk, 0)
    for par in range(2):
        pltpu.make_async_copy(uv.at[par], degsh.at[cixs.at[0]],
                              sems[par]).wait()

    plsc.subcore_barrier()
    pltpu.sync_copy(degsh.at[pl.ds(s * ZB, ZB)], deg_out.at[c, pl.ds(s * ZB, ZB)])


@functools.partial(
    pl.kernel,
    out_type=jax.ShapeDtypeStruct((NC, NPAD, D), jnp.float32),
    mesh=_mesh,
    compiler_params=pltpu.CompilerParams(needs_layout_passes=False),
    scratch_types=(
        pltpu.VMEM((3, CH), jnp.int32),
        pltpu.VMEM((3, CH), jnp.int32),
        pltpu.VMEM((3, CH), jnp.float32),
        pltpu.VMEM((CH,), jnp.float32),
        pltpu.VMEM((NPAD,), jnp.float32),
        pltpu.VMEM((3, CH, D), jnp.float32),
        pltpu.VMEM_SHARED((NPAD, D), jnp.float32),
        pltpu.SemaphoreType.DMA,
        pltpu.SemaphoreType.DMA,
        pltpu.SemaphoreType.DMA,
        pltpu.SemaphoreType.DMA,
        pltpu.SemaphoreType.DMA,
        pltpu.SemaphoreType.DMA,
        pltpu.SemaphoreType.DMA,
        pltpu.SemaphoreType.DMA,
        pltpu.SemaphoreType.DMA,
    ),
)
def _sc_aggregate(x, att3, row3, col3, dr, acc_out, rixs, cixs, atts,
                  normv, drtab, xr, accsh,
                  sg0, sg1, sg2, ss0, ss1, ss2, si0, si1, si2):
    c = lax.axis_index("c")
    s = lax.axis_index("s")
    w = s * NC + c

    pltpu.sync_copy(dr, drtab)

    def zrow(e, _):
        for j in range(D // 16):
            xr[0, e, pl.ds(16 * j, 16)] = jnp.zeros((16,), jnp.float32)
        return 0
    lax.fori_loop(0, CH, zrow, 0)
    for k in range(ZB // CH):
        pltpu.sync_copy(xr.at[0], accsh.at[pl.ds(s * ZB + k * CH, CH)])
    plsc.subcore_barrier()

    sg = (sg0, sg1, sg2)
    ss = (ss0, ss1, ss2)
    si = (si0, si1, si2)

    def issue_idx(g, b):
        pltpu.async_copy(row3.at[w, g], rixs.at[b], si[b])
        pltpu.async_copy(col3.at[w, g], cixs.at[b], si[b])
        pltpu.async_copy(att3.at[w, g], atts.at[b], si[b])

    def wait_idx(g, b):
        pltpu.make_async_copy(row3.at[w, g], rixs.at[b], si[b]).wait()
        pltpu.make_async_copy(col3.at[w, g], cixs.at[b], si[b]).wait()
        pltpu.make_async_copy(att3.at[w, g], atts.at[b], si[b]).wait()

    issue_idx(0, 0)
    issue_idx(1, 1)
    wait_idx(0, 0)
    pltpu.async_copy(x.at[rixs.at[0]], xr.at[0], sg[0])

    def chunk(g, _):
        for par in range(3):
            @pl.when(lax.rem(g, 3) == par)
            def _():
                nb = (par + 1) % 3
                @pl.when(g + 1 < NCH)
                def _():
                    @pl.when(g >= 2)
                    def _():
                        pltpu.make_async_copy(
                            xr.at[nb], accsh.at[cixs.at[nb]], ss[nb]).wait()
                    wait_idx(g + 1, nb)
                    pltpu.async_copy(x.at[rixs.at[nb]], xr.at[nb], sg[nb])

                xb = xr.at[par]
                pltpu.make_async_copy(x.at[rixs.at[par]], xb, sg[par]).wait()

                def grp(i, _):
                    sl = pl.ds(i * 16, 16)
                    r16 = rixs[par, sl]
                    n16 = plsc.load_gather(drtab, [r16]) * atts[par, sl]
                    normv[sl] = n16
                    return 0
                lax.fori_loop(0, CH // 16, grp, 0)

                def scale(e, _):
                    eidx = jnp.zeros((16,), jnp.int32) + e
                    spl = plsc.load_gather(normv, [eidx])
                    for j in range(D // 16):
                        csl = pl.ds(16 * j, 16)
                        xb[e, csl] = xb[e, csl] * spl
                    return 0
                lax.fori_loop(0, CH, scale, 0)

                pltpu.async_copy(xb, accsh.at[cixs.at[par]], ss[par], add=True)

                @pl.when(g + 2 < NCH)
                def _():
                    issue_idx(g + 2, (par + 2) % 3)
        return 0
    lax.fori_loop(0, NCH, chunk, 0)
    for par in range(3):
        pltpu.make_async_copy(xr.at[par], accsh.at[cixs.at[par]],
                              ss[par]).wait()

    plsc.subcore_barrier()
    pltpu.sync_copy(accsh.at[pl.ds(s * ZB, ZB)], acc_out.at[c, pl.ds(s * ZB, ZB)])


def _norm_body(x_ref, o_ref):
    xb = x_ref[...]
    ss = jnp.sum(xb * xb, axis=1, keepdims=True)
    o_ref[...] = xb * lax.rsqrt(jnp.maximum(ss, 1e-12))


def _normalize(x):
    return pl.pallas_call(
        _norm_body,
        grid=(10,),
        in_specs=[pl.BlockSpec((N // 10, D), lambda i: (i, 0))],
        out_specs=pl.BlockSpec((N // 10, D), lambda i: (i, 0)),
        out_shape=jax.ShapeDtypeStruct((N, D), jnp.float32),
    )(x)


def _dr_body(rs_ref, dg_ref, dr_ref):
    ir = 1.0 / jnp.maximum(rs_ref[0] + rs_ref[1], 1e-12)
    dr_ref[...] = lax.rsqrt(dg_ref[0] + dg_ref[1] + 1.0) * ir


def _vec_dr(rsparts, degparts):
    return pl.pallas_call(
        _dr_body,
        out_shape=jax.ShapeDtypeStruct((NPAD // D, D), jnp.float32),
    )(rsparts.reshape(NC, NPAD // D, D), degparts.reshape(NC, NPAD // D, D))


def _mm_body(a0_ref, a1_ref, x_ref, d0_ref, d1_ref, w_ref, b_ref, o_ref,
             on_ref, *, relu):
    dv = lax.rsqrt(d0_ref[...] + d1_ref[...] + 1.0)
    a = (a0_ref[...] + a1_ref[...] + x_ref[...] * dv) * dv
    h = jnp.dot(a, w_ref[...], preferred_element_type=jnp.float32) + b_ref[...]
    if relu:
        h = jnp.maximum(h, 0.0)
    o_ref[...] = h
    if on_ref is not None:
        ss = jnp.sum(h * h, axis=1, keepdims=True)
        on_ref[...] = h * lax.rsqrt(jnp.maximum(ss, 1e-12))


def _mm(a0, a1, xpad, degparts, w, b, relu, with_norm):
    blk = NPAD // 10
    rowspec = pl.BlockSpec((blk, D), lambda i: (i, 0))
    colspec = pl.BlockSpec((blk, 1), lambda i: (i, 0))
    out_shape = jax.ShapeDtypeStruct((NPAD, D), jnp.float32)
    body = _mm_body if with_norm else (
        lambda *refs, relu: _mm_body(*refs, None, relu=relu))
    return pl.pallas_call(
        functools.partial(body, relu=relu),
        grid=(10,),
        in_specs=[
            rowspec, rowspec, rowspec, colspec, colspec,
            pl.BlockSpec((D, D), lambda i: (0, 0)),
            pl.BlockSpec((1, D), lambda i: (0, 0)),
        ],
        out_specs=(rowspec, rowspec) if with_norm else rowspec,
        out_shape=(out_shape, out_shape) if with_norm else out_shape,
    )(a0, a1, xpad, degparts[0].reshape(NPAD, 1), degparts[1].reshape(NPAD, 1),
      w, b)


def _layer(x, xn, row3, col3, w_mat, b_vec, relu, with_norm):
    att3, rsparts = _sc_attention(xn, row3, col3)
    degparts = _sc_degree(att3, row3, col3, rsparts)
    dr = _vec_dr(rsparts, degparts)
    accparts = _sc_aggregate(x, att3, row3, col3, dr.reshape(NPAD))
    xpad = jnp.pad(x, ((0, NPAD - N), (0, 0)))
    return _mm(accparts[0], accparts[1], xpad, degparts,
               w_mat, b_vec.reshape(1, D), relu, with_norm)


def kernel(x, edge_index, W1, b1, W2, b2):
    row3 = edge_index[0].astype(jnp.int32).reshape(NW, NCH, CH)
    col3 = edge_index[1].astype(jnp.int32).reshape(NW, NCH, CH)
    xn = _normalize(x)
    h1, h1n = _layer(x, xn, row3, col3, W1, b1, relu=True, with_norm=True)
    out = _layer(h1[:N], h1n[:N], row3, col3, W2, b2, relu=False,
                 with_norm=False)
    return out[:N]


# speedup vs baseline: 1.0013x; 1.0013x over previous
import functools

import jax
import jax.numpy as jnp
from jax import lax
from jax.experimental import pallas as pl
from jax.experimental.pallas import tpu as pltpu
from jax.experimental.pallas import tpu_sc as plsc

N = 10000
E = 320000
D = 128
THRESH = 0.1
NC = 2
NS = 16
NW = NC * NS
EPW = E // NW
CH = 80
NCH = EPW // CH
NPAD = 10240
ZB = NPAD // NS

_mesh = plsc.VectorSubcoreMesh(core_axis_name="c", subcore_axis_name="s")


def _zero_vec(ref, nwords):
    def body(i, _):
        ref[pl.ds(i * 16, 16)] = jnp.zeros((16,), jnp.float32)
        return 0
    lax.fori_loop(0, nwords // 16, body, 0)


@functools.partial(
    pl.kernel,
    out_type=(
        jax.ShapeDtypeStruct((NW, NCH, CH), jnp.float32),
        jax.ShapeDtypeStruct((NC, NPAD), jnp.float32),
    ),
    mesh=_mesh,
    compiler_params=pltpu.CompilerParams(needs_layout_passes=False),
    scratch_types=(
        pltpu.VMEM((NCH, CH), jnp.int32),
        pltpu.VMEM((NCH, CH), jnp.int32),
        pltpu.VMEM((2, CH), jnp.float32),
        pltpu.VMEM((2, CH, D), jnp.float32),
        pltpu.VMEM((2, CH, D), jnp.float32),
        pltpu.VMEM((16, 17), jnp.float32),
        pltpu.VMEM((ZB,), jnp.float32),
        pltpu.VMEM_SHARED((NPAD,), jnp.float32),
        pltpu.SemaphoreType.DMA,
        pltpu.SemaphoreType.DMA,
        pltpu.SemaphoreType.DMA,
        pltpu.SemaphoreType.DMA,
        pltpu.SemaphoreType.DMA,
        pltpu.SemaphoreType.DMA,
        pltpu.SemaphoreType.DMA,
        pltpu.SemaphoreType.DMA,
    ),
)
def _sc_attention(xn, row3, col3, att_out, rs_out, rixs, cixs, attv,
                  arows, brows, tbuf, zbuf, rssh, sa0, sb0, sa1, sb1, so0, so1,
                  ss0, ss1):
    c = lax.axis_index("c")
    s = lax.axis_index("s")
    w = s * NC + c

    _zero_vec(zbuf, ZB)
    pltpu.sync_copy(zbuf, rssh.at[pl.ds(s * ZB, ZB)])
    plsc.subcore_barrier()

    pltpu.sync_copy(row3.at[w], rixs)
    pltpu.sync_copy(col3.at[w], cixs)

    lanes = lax.iota(jnp.int32, 16)
    sems = ((sa0, sb0, so0, ss0), (sa1, sb1, so1, ss1))

    def issue(g, b):
        pltpu.async_copy(xn.at[rixs.at[g]], arows.at[b], sems[b][0])
        pltpu.async_copy(xn.at[cixs.at[g]], brows.at[b], sems[b][1])

    issue(0, 0)

    def do_chunk(g, b):
        ar = arows.at[b]
        br = brows.at[b]
        ab = attv.at[b]
        pltpu.make_async_copy(xn.at[rixs.at[g]], ar, sems[b][0]).wait()
        pltpu.make_async_copy(xn.at[cixs.at[g]], br, sems[b][1]).wait()

        def grp16(i, _):
            for l in range(16):
                e = i * 16 + l
                acc = ar[e, pl.ds(0, 16)] * br[e, pl.ds(0, 16)]
                for j in range(1, 8):
                    acc = acc + ar[e, pl.ds(16 * j, 16)] * br[e, pl.ds(16 * j, 16)]
                tbuf[l, pl.ds(0, 16)] = acc
            cols = [plsc.load_gather(tbuf, [lanes, jnp.full((16,), j, jnp.int32)])
                    for j in range(16)]
            while len(cols) > 1:
                cols = [cols[k] + cols[k + 1] for k in range(0, len(cols), 2)]
            v = cols[0]
            ab[pl.ds(i * 16, 16)] = jnp.where(v < THRESH, 0.0, v)
            return 0
        lax.fori_loop(0, CH // 16, grp16, 0)

        pltpu.async_copy(ab, att_out.at[w, g], sems[b][2])
        pltpu.async_copy(ab, rssh.at[rixs.at[g]], sems[b][3], add=True)

    def body(g, _):
        for par in range(2):
            @pl.when(lax.rem(g, 2) == par)
            def _():
                @pl.when(g + 1 < NCH)
                def _():
                    issue(g + 1, 1 - par)
                @pl.when(g >= 2)
                def _():
                    pltpu.make_async_copy(
                        attv.at[par], att_out.at[w, g], sems[par][2]).wait()
                    pltpu.make_async_copy(
                        attv.at[par], rssh.at[rixs.at[g]], sems[par][3]).wait()
                do_chunk(g, par)
        return 0
    lax.fori_loop(0, NCH, body, 0)
    for par in range(2):
        pltpu.make_async_copy(attv.at[par], att_out.at[w, 0],
                              sems[par][2]).wait()
        pltpu.make_async_copy(attv.at[par], rssh.at[rixs.at[0]],
                              sems[par][3]).wait()

    plsc.subcore_barrier()
    pltpu.sync_copy(rssh.at[pl.ds(s * ZB, ZB)], rs_out.at[c, pl.ds(s * ZB, ZB)])


@functools.partial(
    pl.kernel,
    out_type=jax.ShapeDtypeStruct((NC, NPAD), jnp.float32),
    mesh=_mesh,
    compiler_params=pltpu.CompilerParams(needs_layout_passes=False),
    scratch_types=(
        pltpu.VMEM((NCH, CH), jnp.int32),
        pltpu.VMEM((NCH, CH), jnp.int32),
        pltpu.VMEM((NCH, CH), jnp.float32),
        pltpu.VMEM((2, CH), jnp.float32),
        pltpu.VMEM((NPAD,), jnp.float32),
        pltpu.VMEM((NPAD,), jnp.float32),
        pltpu.VMEM((ZB,), jnp.float32),
        pltpu.VMEM_SHARED((NPAD,), jnp.float32),
        pltpu.SemaphoreType.DMA,
        pltpu.SemaphoreType.DMA,
    ),
)
def _sc_degree(att3, row3, col3, rsparts, deg_out, rixs, cixs, atts, uv,
               irtab, rstmp, zbuf, degsh, su0, su1):
    c = lax.axis_index("c")
    s = lax.axis_index("s")
    w = s * NC + c

    pltpu.sync_copy(rsparts.at[0], irtab)
    pltpu.sync_copy(rsparts.at[1], rstmp)

    def mkir(i, _):
        sl = pl.ds(i * 16, 16)
        rs = irtab[sl] + rstmp[sl]
        irtab[sl] = 1.0 / jnp.maximum(rs, 1e-12)
        return 0
    lax.fori_loop(0, NPAD // 16, mkir, 0)

    _zero_vec(zbuf, ZB)
    pltpu.sync_copy(zbuf, degsh.at[pl.ds(s * ZB, ZB)])
    pltpu.sync_copy(row3.at[w], rixs)
    pltpu.sync_copy(col3.at[w], cixs)
    pltpu.sync_copy(att3.at[w], atts)
    plsc.subcore_barrier()

    sems = (su0, su1)

    def chunk(g, _):
        for par in range(2):
            @pl.when(lax.rem(g, 2) == par)
            def _():
                ub = uv.at[par]
                @pl.when(g >= 2)
                def _():
                    pltpu.make_async_copy(
                        ub, degsh.at[cixs.at[g]], sems[par]).wait()

                def grp(i, _):
                    sl = pl.ds(i * 16, 16)
                    r16 = rixs[g, sl]
                    ir = plsc.load_gather(irtab, [r16])
                    ub[sl] = atts[g, sl] * ir
                    return 0
                lax.fori_loop(0, CH // 16, grp, 0)

                pltpu.async_copy(ub, degsh.at[cixs.at[g]], sems[par], add=True)
        return 0
    lax.fori_loop(0, NCH, chunk, 0)
    for par in range(2):
        pltpu.make_async_copy(uv.at[par], degsh.at[cixs.at[0]],
                              sems[par]).wait()

    plsc.subcore_barrier()
    pltpu.sync_copy(degsh.at[pl.ds(s * ZB, ZB)], deg_out.at[c, pl.ds(s * ZB, ZB)])


@functools.partial(
    pl.kernel,
    out_type=jax.ShapeDtypeStruct((NC, NPAD, D), jnp.float32),
    mesh=_mesh,
    compiler_params=pltpu.CompilerParams(needs_layout_passes=False),
    scratch_types=(
        pltpu.VMEM((3, CH), jnp.int32),
        pltpu.VMEM((3, CH), jnp.int32),
        pltpu.VMEM((3, CH), jnp.float32),
        pltpu.VMEM((CH,), jnp.float32),
        pltpu.VMEM((NPAD,), jnp.float32),
        pltpu.VMEM((3, CH, D), jnp.float32),
        pltpu.VMEM_SHARED((NPAD, D), jnp.float32),
        pltpu.SemaphoreType.DMA,
        pltpu.SemaphoreType.DMA,
        pltpu.SemaphoreType.DMA,
        pltpu.SemaphoreType.DMA,
        pltpu.SemaphoreType.DMA,
        pltpu.SemaphoreType.DMA,
        pltpu.SemaphoreType.DMA,
        pltpu.SemaphoreType.DMA,
        pltpu.SemaphoreType.DMA,
    ),
)
def _sc_aggregate(x, att3, row3, col3, dr, acc_out, rixs, cixs, atts,
                  normv, drtab, xr, accsh,
                  sg0, sg1, sg2, ss0, ss1, ss2, si0, si1, si2):
    c = lax.axis_index("c")
    s = lax.axis_index("s")
    w = s * NC + c

    pltpu.sync_copy(dr, drtab)

    def zrow(e, _):
        for j in range(D // 16):
            xr[0, e, pl.ds(16 * j, 16)] = jnp.zeros((16,), jnp.float32)
        return 0
    lax.fori_loop(0, CH, zrow, 0)
    for k in range(ZB // CH):
        pltpu.sync_copy(xr.at[0], accsh.at[pl.ds(s * ZB + k * CH, CH)])
    plsc.subcore_barrier()

    sg = (sg0, sg1, sg2)
    ss = (ss0, ss1, ss2)
    si = (si0, si1, si2)

    def issue_idx(g, b):
        pltpu.async_copy(row3.at[w, g], rixs.at[b], si[b])
        pltpu.async_copy(col3.at[w, g], cixs.at[b], si[b])
        pltpu.async_copy(att3.at[w, g], atts.at[b], si[b])

    def wait_idx(g, b):
        pltpu.make_async_copy(row3.at[w, g], rixs.at[b], si[b]).wait()
        pltpu.make_async_copy(col3.at[w, g], cixs.at[b], si[b]).wait()
        pltpu.make_async_copy(att3.at[w, g], atts.at[b], si[b]).wait()

    issue_idx(0, 0)
    issue_idx(1, 1)
    wait_idx(0, 0)
    pltpu.async_copy(x.at[rixs.at[0]], xr.at[0], sg[0])

    def chunk(g, _):
        for par in range(3):
            @pl.when(lax.rem(g, 3) == par)
            def _():
                nb = (par + 1) % 3
                @pl.when(g + 1 < NCH)
                def _():
                    @pl.when(g >= 2)
                    def _():
                        pltpu.make_async_copy(
                            xr.at[nb], accsh.at[cixs.at[nb]], ss[nb]).wait()
                    wait_idx(g + 1, nb)
                    pltpu.async_copy(x.at[rixs.at[nb]], xr.at[nb], sg[nb])

                xb = xr.at[par]
                pltpu.make_async_copy(x.at[rixs.at[par]], xb, sg[par]).wait()

                def grp(i, _):
                    sl = pl.ds(i * 16, 16)
                    r16 = rixs[par, sl]
                    n16 = plsc.load_gather(drtab, [r16]) * atts[par, sl]
                    normv[sl] = n16
                    return 0
                lax.fori_loop(0, CH // 16, grp, 0)

                def scale(e, _):
                    eidx = jnp.zeros((16,), jnp.int32) + e
                    spl = plsc.load_gather(normv, [eidx])
                    for j in range(D // 16):
                        csl = pl.ds(16 * j, 16)
                        xb[e, csl] = xb[e, csl] * spl
                    return 0
                lax.fori_loop(0, CH, scale, 0)

                pltpu.async_copy(xb, accsh.at[cixs.at[par]], ss[par], add=True)

                @pl.when(g + 2 < NCH)
                def _():
                    issue_idx(g + 2, (par + 2) % 3)
        return 0
    lax.fori_loop(0, NCH, chunk, 0)
    for par in range(3):
        pltpu.make_async_copy(xr.at[par], accsh.at[cixs.at[par]],
                              ss[par]).wait()

    plsc.subcore_barrier()
    pltpu.sync_copy(accsh.at[pl.ds(s * ZB, ZB)], acc_out.at[c, pl.ds(s * ZB, ZB)])


def _norm_body(x_ref, o_ref):
    xb = x_ref[...]
    ss = jnp.sum(xb * xb, axis=1, keepdims=True)
    o_ref[...] = xb * lax.rsqrt(jnp.maximum(ss, 1e-12))


def _normalize(x):
    return pl.pallas_call(
        _norm_body,
        grid=(10,),
        in_specs=[pl.BlockSpec((NPAD // 10, D), lambda i: (i, 0))],
        out_specs=pl.BlockSpec((NPAD // 10, D), lambda i: (i, 0)),
        out_shape=jax.ShapeDtypeStruct((NPAD, D), jnp.float32),
    )(x)


def _dr_body(rs_ref, dg_ref, dr_ref):
    ir = 1.0 / jnp.maximum(rs_ref[0] + rs_ref[1], 1e-12)
    dr_ref[...] = lax.rsqrt(dg_ref[0] + dg_ref[1] + 1.0) * ir


def _vec_dr(rsparts, degparts):
    return pl.pallas_call(
        _dr_body,
        out_shape=jax.ShapeDtypeStruct((NPAD // D, D), jnp.float32),
    )(rsparts.reshape(NC, NPAD // D, D), degparts.reshape(NC, NPAD // D, D))


def _mm_body(a0_ref, a1_ref, x_ref, d0_ref, d1_ref, w_ref, b_ref, o_ref,
             on_ref, *, relu):
    dv = lax.rsqrt(d0_ref[...] + d1_ref[...] + 1.0)
    a = (a0_ref[...] + a1_ref[...] + x_ref[...] * dv) * dv
    h = jnp.dot(a, w_ref[...], preferred_element_type=jnp.float32) + b_ref[...]
    if relu:
        h = jnp.maximum(h, 0.0)
    o_ref[...] = h
    if on_ref is not None:
        ss = jnp.sum(h * h, axis=1, keepdims=True)
        on_ref[...] = h * lax.rsqrt(jnp.maximum(ss, 1e-12))


def _mm(a0, a1, xpad, degparts, w, b, relu, with_norm):
    blk = NPAD // 10
    rowspec = pl.BlockSpec((blk, D), lambda i: (i, 0))
    colspec = pl.BlockSpec((blk, 1), lambda i: (i, 0))
    out_shape = jax.ShapeDtypeStruct((NPAD, D), jnp.float32)
    body = _mm_body if with_norm else (
        lambda *refs, relu: _mm_body(*refs, None, relu=relu))
    return pl.pallas_call(
        functools.partial(body, relu=relu),
        grid=(10,),
        in_specs=[
            rowspec, rowspec, rowspec, colspec, colspec,
            pl.BlockSpec((D, D), lambda i: (0, 0)),
            pl.BlockSpec((1, D), lambda i: (0, 0)),
        ],
        out_specs=(rowspec, rowspec) if with_norm else rowspec,
        out_shape=(out_shape, out_shape) if with_norm else out_shape,
    )(a0, a1, xpad, degparts[0].reshape(NPAD, 1), degparts[1].reshape(NPAD, 1),
      w, b)


def _layer(x, xn, row3, col3, w_mat, b_vec, relu, with_norm):
    att3, rsparts = _sc_attention(xn, row3, col3)
    degparts = _sc_degree(att3, row3, col3, rsparts)
    dr = _vec_dr(rsparts, degparts)
    accparts = _sc_aggregate(x, att3, row3, col3, dr.reshape(NPAD))
    return _mm(accparts[0], accparts[1], x, degparts,
               w_mat, b_vec.reshape(1, D), relu, with_norm)


def kernel(x, edge_index, W1, b1, W2, b2):
    row3 = edge_index[0].astype(jnp.int32).reshape(NW, NCH, CH)
    col3 = edge_index[1].astype(jnp.int32).reshape(NW, NCH, CH)
    xpad = jnp.pad(x, ((0, NPAD - N), (0, 0)))
    xn = _normalize(xpad)
    h1, h1n = _layer(xpad, xn, row3, col3, W1, b1, relu=True, with_norm=True)
    out = _layer(h1, h1n, row3, col3, W2, b2, relu=False, with_norm=False)
    return out[:N]


# speedup vs baseline: 1.0380x; 1.0366x over previous
import functools

import jax
import jax.numpy as jnp
from jax import lax
from jax.experimental import pallas as pl
from jax.experimental.pallas import tpu as pltpu
from jax.experimental.pallas import tpu_sc as plsc

N = 10000
E = 320000
D = 128
THRESH = 0.1
NC = 2
NS = 16
NW = NC * NS
EPW = E // NW
CH = 80
NCH = EPW // CH
NPAD = 10240
ZB = NPAD // NS

_mesh = plsc.VectorSubcoreMesh(core_axis_name="c", subcore_axis_name="s")


def _zero_vec(ref, nwords):
    def body(i, _):
        ref[pl.ds(i * 16, 16)] = jnp.zeros((16,), jnp.float32)
        return 0
    lax.fori_loop(0, nwords // 16, body, 0)


@functools.partial(
    pl.kernel,
    out_type=(
        jax.ShapeDtypeStruct((NW, NCH, CH), jnp.float32),
        jax.ShapeDtypeStruct((NC, NPAD), jnp.float32),
    ),
    mesh=_mesh,
    compiler_params=pltpu.CompilerParams(needs_layout_passes=False),
    scratch_types=(
        pltpu.VMEM((NCH, CH), jnp.int32),
        pltpu.VMEM((NCH, CH), jnp.int32),
        pltpu.VMEM((2, CH), jnp.float32),
        pltpu.VMEM((2, CH, D), jnp.float32),
        pltpu.VMEM((2, CH, D), jnp.float32),
        pltpu.VMEM((16, 17), jnp.float32),
        pltpu.VMEM((ZB,), jnp.float32),
        pltpu.VMEM_SHARED((NPAD,), jnp.float32),
        pltpu.SemaphoreType.DMA,
        pltpu.SemaphoreType.DMA,
        pltpu.SemaphoreType.DMA,
        pltpu.SemaphoreType.DMA,
        pltpu.SemaphoreType.DMA,
        pltpu.SemaphoreType.DMA,
        pltpu.SemaphoreType.DMA,
        pltpu.SemaphoreType.DMA,
    ),
)
def _sc_attention(xn, row3, col3, att_out, rs_out, rixs, cixs, attv,
                  arows, brows, tbuf, zbuf, rssh, sa0, sb0, sa1, sb1, so0, so1,
                  ss0, ss1):
    c = lax.axis_index("c")
    s = lax.axis_index("s")
    w = s * NC + c

    _zero_vec(zbuf, ZB)
    pltpu.sync_copy(zbuf, rssh.at[pl.ds(s * ZB, ZB)])
    plsc.subcore_barrier()

    pltpu.sync_copy(row3.at[w], rixs)
    pltpu.sync_copy(col3.at[w], cixs)

    lanes = lax.iota(jnp.int32, 16)
    sems = ((sa0, sb0, so0, ss0), (sa1, sb1, so1, ss1))

    def issue(g, b):
        pltpu.async_copy(xn.at[rixs.at[g]], arows.at[b], sems[b][0])
        pltpu.async_copy(xn.at[cixs.at[g]], brows.at[b], sems[b][1])

    issue(0, 0)

    def do_chunk(g, b):
        ar = arows.at[b]
        br = brows.at[b]
        ab = attv.at[b]
        pltpu.make_async_copy(xn.at[rixs.at[g]], ar, sems[b][0]).wait()
        pltpu.make_async_copy(xn.at[cixs.at[g]], br, sems[b][1]).wait()

        def grp16(i, _):
            for l in range(16):
                e = i * 16 + l
                acc = ar[e, pl.ds(0, 16)] * br[e, pl.ds(0, 16)]
                for j in range(1, 8):
                    acc = acc + ar[e, pl.ds(16 * j, 16)] * br[e, pl.ds(16 * j, 16)]
                tbuf[l, pl.ds(0, 16)] = acc
            cols = [plsc.load_gather(tbuf, [lanes, jnp.full((16,), j, jnp.int32)])
                    for j in range(16)]
            while len(cols) > 1:
                cols = [cols[k] + cols[k + 1] for k in range(0, len(cols), 2)]
            v = cols[0]
            ab[pl.ds(i * 16, 16)] = jnp.where(v < THRESH, 0.0, v)
            return 0
        lax.fori_loop(0, CH // 16, grp16, 0)

        pltpu.async_copy(ab, att_out.at[w, g], sems[b][2])
        pltpu.async_copy(ab, rssh.at[rixs.at[g]], sems[b][3], add=True)

    def body(g, _):
        for par in range(2):
            @pl.when(lax.rem(g, 2) == par)
            def _():
                @pl.when(g + 1 < NCH)
                def _():
                    issue(g + 1, 1 - par)
                @pl.when(g >= 2)
                def _():
                    pltpu.make_async_copy(
                        attv.at[par], att_out.at[w, g], sems[par][2]).wait()
                    pltpu.make_async_copy(
                        attv.at[par], rssh.at[rixs.at[g]], sems[par][3]).wait()
                do_chunk(g, par)
        return 0
    lax.fori_loop(0, NCH, body, 0)
    for par in range(2):
        pltpu.make_async_copy(attv.at[par], att_out.at[w, 0],
                              sems[par][2]).wait()
        pltpu.make_async_copy(attv.at[par], rssh.at[rixs.at[0]],
                              sems[par][3]).wait()

    plsc.subcore_barrier()
    pltpu.sync_copy(rssh.at[pl.ds(s * ZB, ZB)], rs_out.at[c, pl.ds(s * ZB, ZB)])


@functools.partial(
    pl.kernel,
    out_type=jax.ShapeDtypeStruct((NC, NPAD), jnp.float32),
    mesh=_mesh,
    compiler_params=pltpu.CompilerParams(needs_layout_passes=False),
    scratch_types=(
        pltpu.VMEM((NCH, CH), jnp.int32),
        pltpu.VMEM((NCH, CH), jnp.int32),
        pltpu.VMEM((NCH, CH), jnp.float32),
        pltpu.VMEM((2, CH), jnp.float32),
        pltpu.VMEM((NPAD,), jnp.float32),
        pltpu.VMEM((ZB,), jnp.float32),
        pltpu.VMEM_SHARED((NPAD,), jnp.float32),
        pltpu.SemaphoreType.DMA,
        pltpu.SemaphoreType.DMA,
    ),
)
def _sc_degree(att3, row3, col3, invrow, deg_out, rixs, cixs, atts, uv,
               irtab, zbuf, degsh, su0, su1):
    c = lax.axis_index("c")
    s = lax.axis_index("s")
    w = s * NC + c

    pltpu.sync_copy(invrow, irtab)
    _zero_vec(zbuf, ZB)
    pltpu.sync_copy(zbuf, degsh.at[pl.ds(s * ZB, ZB)])
    pltpu.sync_copy(row3.at[w], rixs)
    pltpu.sync_copy(col3.at[w], cixs)
    pltpu.sync_copy(att3.at[w], atts)
    plsc.subcore_barrier()

    sems = (su0, su1)

    def chunk(g, _):
        for par in range(2):
            @pl.when(lax.rem(g, 2) == par)
            def _():
                ub = uv.at[par]
                @pl.when(g >= 2)
                def _():
                    pltpu.make_async_copy(
                        ub, degsh.at[cixs.at[g]], sems[par]).wait()

                def grp(i, _):
                    sl = pl.ds(i * 16, 16)
                    r16 = rixs[g, sl]
                    ir = plsc.load_gather(irtab, [r16])
                    ub[sl] = atts[g, sl] * ir
                    return 0
                lax.fori_loop(0, CH // 16, grp, 0)

                pltpu.async_copy(ub, degsh.at[cixs.at[g]], sems[par], add=True)
        return 0
    lax.fori_loop(0, NCH, chunk, 0)
    for par in range(2):
        pltpu.make_async_copy(uv.at[par], degsh.at[cixs.at[0]],
                              sems[par]).wait()

    plsc.subcore_barrier()
    pltpu.sync_copy(degsh.at[pl.ds(s * ZB, ZB)], deg_out.at[c, pl.ds(s * ZB, ZB)])


@functools.partial(
    pl.kernel,
    out_type=jax.ShapeDtypeStruct((NC, NPAD, D), jnp.float32),
    mesh=_mesh,
    compiler_params=pltpu.CompilerParams(needs_layout_passes=False),
    scratch_types=(
        pltpu.VMEM((3, CH), jnp.int32),
        pltpu.VMEM((3, CH), jnp.int32),
        pltpu.VMEM((3, CH), jnp.float32),
        pltpu.VMEM((CH,), jnp.float32),
        pltpu.VMEM((NPAD,), jnp.float32),
        pltpu.VMEM((3, CH, D), jnp.float32),
        pltpu.VMEM_SHARED((NPAD, D), jnp.float32),
        pltpu.SemaphoreType.DMA,
        pltpu.SemaphoreType.DMA,
        pltpu.SemaphoreType.DMA,
        pltpu.SemaphoreType.DMA,
        pltpu.SemaphoreType.DMA,
        pltpu.SemaphoreType.DMA,
        pltpu.SemaphoreType.DMA,
        pltpu.SemaphoreType.DMA,
        pltpu.SemaphoreType.DMA,
    ),
)
def _sc_aggregate(x, att3, row3, col3, dr, acc_out, rixs, cixs, atts,
                  normv, drtab, xr, accsh,
                  sg0, sg1, sg2, ss0, ss1, ss2, si0, si1, si2):
    c = lax.axis_index("c")
    s = lax.axis_index("s")
    w = s * NC + c

    pltpu.sync_copy(dr, drtab)

    def zrow(e, _):
        for j in range(D // 16):
            xr[0, e, pl.ds(16 * j, 16)] = jnp.zeros((16,), jnp.float32)
        return 0
    lax.fori_loop(0, CH, zrow, 0)
    for k in range(ZB // CH):
        pltpu.sync_copy(xr.at[0], accsh.at[pl.ds(s * ZB + k * CH, CH)])
    plsc.subcore_barrier()

    sg = (sg0, sg1, sg2)
    ss = (ss0, ss1, ss2)
    si = (si0, si1, si2)

    def issue_idx(g, b):
        pltpu.async_copy(row3.at[w, g], rixs.at[b], si[b])
        pltpu.async_copy(col3.at[w, g], cixs.at[b], si[b])
        pltpu.async_copy(att3.at[w, g], atts.at[b], si[b])

    def wait_idx(g, b):
        pltpu.make_async_copy(row3.at[w, g], rixs.at[b], si[b]).wait()
        pltpu.make_async_copy(col3.at[w, g], cixs.at[b], si[b]).wait()
        pltpu.make_async_copy(att3.at[w, g], atts.at[b], si[b]).wait()

    issue_idx(0, 0)
    issue_idx(1, 1)
    wait_idx(0, 0)
    pltpu.async_copy(x.at[rixs.at[0]], xr.at[0], sg[0])

    def chunk(g, _):
        for par in range(3):
            @pl.when(lax.rem(g, 3) == par)
            def _():
                nb = (par + 1) % 3
                @pl.when(g + 1 < NCH)
                def _():
                    @pl.when(g >= 2)
                    def _():
                        pltpu.make_async_copy(
                            xr.at[nb], accsh.at[cixs.at[nb]], ss[nb]).wait()
                    wait_idx(g + 1, nb)
                    pltpu.async_copy(x.at[rixs.at[nb]], xr.at[nb], sg[nb])

                xb = xr.at[par]
                pltpu.make_async_copy(x.at[rixs.at[par]], xb, sg[par]).wait()

                def grp(i, _):
                    sl = pl.ds(i * 16, 16)
                    r16 = rixs[par, sl]
                    n16 = plsc.load_gather(drtab, [r16]) * atts[par, sl]
                    normv[sl] = n16
                    return 0
                lax.fori_loop(0, CH // 16, grp, 0)

                def scale(e, _):
                    eidx = jnp.zeros((16,), jnp.int32) + e
                    spl = plsc.load_gather(normv, [eidx])
                    for j in range(D // 16):
                        csl = pl.ds(16 * j, 16)
                        xb[e, csl] = xb[e, csl] * spl
                    return 0
                lax.fori_loop(0, CH, scale, 0, unroll=4)

                pltpu.async_copy(xb, accsh.at[cixs.at[par]], ss[par], add=True)

                @pl.when(g + 2 < NCH)
                def _():
                    issue_idx(g + 2, (par + 2) % 3)
        return 0
    lax.fori_loop(0, NCH, chunk, 0)
    for par in range(3):
        pltpu.make_async_copy(xr.at[par], accsh.at[cixs.at[par]],
                              ss[par]).wait()

    plsc.subcore_barrier()
    pltpu.sync_copy(accsh.at[pl.ds(s * ZB, ZB)], acc_out.at[c, pl.ds(s * ZB, ZB)])


def _norm_body(x_ref, o_ref):
    xb = x_ref[...]
    ss = jnp.sum(xb * xb, axis=1, keepdims=True)
    o_ref[...] = xb * lax.rsqrt(jnp.maximum(ss, 1e-12))


def _normalize(x):
    return pl.pallas_call(
        _norm_body,
        grid=(10,),
        in_specs=[pl.BlockSpec((NPAD // 10, D), lambda i: (i, 0))],
        out_specs=pl.BlockSpec((NPAD // 10, D), lambda i: (i, 0)),
        out_shape=jax.ShapeDtypeStruct((NPAD, D), jnp.float32),
    )(x)


def _invrow_body(p_ref, o_ref):
    o_ref[...] = 1.0 / jnp.maximum(p_ref[0] + p_ref[1], 1e-12)


def _vec_invrow(rsparts):
    return pl.pallas_call(
        _invrow_body,
        out_shape=jax.ShapeDtypeStruct((NPAD // D, D), jnp.float32),
    )(rsparts.reshape(NC, NPAD // D, D))


def _dr_body(ir_ref, dg_ref, dr_ref):
    dr_ref[...] = lax.rsqrt(dg_ref[0] + dg_ref[1] + 1.0) * ir_ref[...]


def _vec_dr(invrow, degparts):
    return pl.pallas_call(
        _dr_body,
        out_shape=jax.ShapeDtypeStruct((NPAD // D, D), jnp.float32),
    )(invrow, degparts.reshape(NC, NPAD // D, D))


def _mm_body(a0_ref, a1_ref, x_ref, d0_ref, d1_ref, w_ref, b_ref, o_ref,
             on_ref, *, relu):
    dv = lax.rsqrt(d0_ref[...] + d1_ref[...] + 1.0)
    a = (a0_ref[...] + a1_ref[...] + x_ref[...] * dv) * dv
    h = jnp.dot(a, w_ref[...], preferred_element_type=jnp.float32) + b_ref[...]
    if relu:
        h = jnp.maximum(h, 0.0)
    o_ref[...] = h
    if on_ref is not None:
        ss = jnp.sum(h * h, axis=1, keepdims=True)
        on_ref[...] = h * lax.rsqrt(jnp.maximum(ss, 1e-12))


def _mm(a0, a1, xpad, degparts, w, b, relu, with_norm):
    blk = NPAD // 10
    rowspec = pl.BlockSpec((blk, D), lambda i: (i, 0))
    colspec = pl.BlockSpec((blk, 1), lambda i: (i, 0))
    out_shape = jax.ShapeDtypeStruct((NPAD, D), jnp.float32)
    body = _mm_body if with_norm else (
        lambda *refs, relu: _mm_body(*refs, None, relu=relu))
    return pl.pallas_call(
        functools.partial(body, relu=relu),
        grid=(10,),
        in_specs=[
            rowspec, rowspec, rowspec, colspec, colspec,
            pl.BlockSpec((D, D), lambda i: (0, 0)),
            pl.BlockSpec((1, D), lambda i: (0, 0)),
        ],
        out_specs=(rowspec, rowspec) if with_norm else rowspec,
        out_shape=(out_shape, out_shape) if with_norm else out_shape,
    )(a0, a1, xpad, degparts[0].reshape(NPAD, 1), degparts[1].reshape(NPAD, 1),
      w, b)


def _layer(x, xn, row3, col3, w_mat, b_vec, relu, with_norm):
    att3, rsparts = _sc_attention(xn, row3, col3)
    invrow = _vec_invrow(rsparts)
    degparts = _sc_degree(att3, row3, col3, invrow.reshape(NPAD))
    dr = _vec_dr(invrow, degparts)
    accparts = _sc_aggregate(x, att3, row3, col3, dr.reshape(NPAD))
    return _mm(accparts[0], accparts[1], x, degparts,
               w_mat, b_vec.reshape(1, D), relu, with_norm)


def kernel(x, edge_index, W1, b1, W2, b2):
    row3 = edge_index[0].astype(jnp.int32).reshape(NW, NCH, CH)
    col3 = edge_index[1].astype(jnp.int32).reshape(NW, NCH, CH)
    xpad = jnp.pad(x, ((0, NPAD - N), (0, 0)))
    xn = _normalize(xpad)
    h1, h1n = _layer(xpad, xn, row3, col3, W1, b1, relu=True, with_norm=True)
    out = _layer(h1, h1n, row3, col3, W2, b2, relu=False, with_norm=False)
    return out[:N]
